# baseline (speedup 1.0000x reference)
import functools

import jax
import jax.numpy as jnp
from jax import lax
from jax.experimental import pallas as pl
from jax.experimental.pallas import tpu as pltpu
from jax.experimental.pallas import tpu_sc as plsc

N_NODES = 100000
DEG = 16
BQ = 16384
D_FEAT = 9
D_H1 = 20
N_OUT = 15

NP = 102400
NC, NS = 2, 16
NW = NC * NS
L = 16

CH = 128
CHA = 128
A_CHUNKS = 28
NSUP = A_CHUNKS // 2
A_SPAN = CHA * A_CHUNKS
A_LAST_START = N_NODES - A_SPAN
A_STRIDE = 3128

QPW = BQ // NW
C_CHUNKS = QPW // CH

f32 = jnp.float32
i32 = jnp.int32

_MESH = plsc.VectorSubcoreMesh(
    core_axis_name="c", subcore_axis_name="s", num_cores=NC, num_subcores=NS)


def _wid():
    return lax.axis_index("s") * NC + lax.axis_index("c")


def _zero_rows(ref, n):
    def body(i, _):
        ref[i] = jnp.zeros((L,), f32)
        return 0
    lax.fori_loop(0, n, body, 0)


@functools.partial(
    pl.kernel,
    out_type=jax.ShapeDtypeStruct((NP, L), f32),
    mesh=_MESH,
    scratch_types=[
        pltpu.VMEM((2, 2, CHA, DEG), i32),
        pltpu.VMEM((2, 2, DEG, CHA), i32),
        pltpu.VMEM((2, 2, CHA, L), f32),
        pltpu.VMEM_SHARED((NP, L), f32),
        pltpu.SemaphoreType.DMA,
        pltpu.SemaphoreType.DMA,
        pltpu.SemaphoreType.DMA,
        pltpu.SemaphoreType.DMA,
        pltpu.SemaphoreType.DMA,
        pltpu.SemaphoreType.DMA,
    ],
    compiler_params=pltpu.CompilerParams(use_tc_tiling_on_sc=False, needs_layout_passes=False),
)
def _agg_kernel(fpad_hbm, neigh_hbm, agg_hbm, nbr_v, idx_v, acc_v, ftab,
                lsem0, lsem1, gsem0, gsem1, ssem0, ssem1):
    lsem = (lsem0, lsem1)
    gsem = (gsem0, gsem1)
    ssem = (ssem0, ssem1)
    wid = _wid()
    start = pl.multiple_of(jnp.minimum(wid * A_STRIDE, A_LAST_START), 8)
    sid = lax.axis_index("s")
    trows = NP // NS
    tstart = pl.multiple_of(sid * trows, 8)
    pltpu.sync_copy(fpad_hbm.at[pl.ds(tstart, trows)],
                    ftab.at[pl.ds(tstart, trows)])
    plsc.subcore_barrier()

    def idx_src(g, h):
        base = pl.multiple_of(start + (2 * g + h) * CHA, 8)
        return neigh_hbm.at[pl.ds(base, CHA)]

    def fire_idx(g, b):
        for h in range(2):
            pltpu.async_copy(idx_src(g, h), nbr_v.at[b, h], lsem[b])

    def drain_idx(g, b):
        for h in range(2):
            pltpu.make_async_copy(
                idx_src(g, h), nbr_v.at[b, h], lsem[b]).wait()
        for h in range(2):
            for j in range(DEG):
                col = jnp.full((L,), j, i32)
                for gg in range(CHA // L):
                    rid = lax.iota(i32, L) + gg * L
                    idx_v[b, h, j, pl.ds(gg * L, L)] = plsc.load_gather(
                        nbr_v.at[b, h], [rid, col])

    def store_dst(g, h):
        base = pl.multiple_of(start + (2 * g + h) * CHA, 8)
        return agg_hbm.at[pl.ds(base, CHA)]

    def super_body(g, b):
        b1 = 1 - b
        drain_idx(g, b)

        @pl.when(g >= 2)
        def _():
            for h in range(2):
                pltpu.make_async_copy(
                    acc_v.at[b, h], store_dst(g - 2, h), ssem[b]).wait()
        for h in range(2):
            def zb(i, _, h=h):
                acc_v[b, h, i] = jnp.zeros((L,), f32)
                return 0
            lax.fori_loop(0, CHA, zb, 0)
        for h in range(2):
            for j in range(DEG):
                pltpu.async_copy(
                    ftab.at[idx_v.at[b, h, j]], acc_v.at[b, h], gsem[b],
                    add=True)

        @pl.when(g >= 1)
        def _():
            for h in range(2):
                for j in range(DEG):
                    pltpu.make_async_copy(
                        agg_hbm.at[pl.ds(0, CHA)], acc_v.at[b1, h],
                        gsem[b1]).wait()
            for h in range(2):
                pltpu.async_copy(acc_v.at[b1, h], store_dst(g - 1, h),
                                 ssem[b1])

        @pl.when(g <= NSUP - 2)
        def _():
            fire_idx(g + 1, b1)

    fire_idx(0, 0)

    def pair_body(k, _):
        super_body(2 * k, 0)
        super_body(2 * k + 1, 1)
        return 0

    lax.fori_loop(0, NSUP // 2, pair_body, 0)
    for h in range(2):
        for j in range(DEG):
            pltpu.make_async_copy(
                agg_hbm.at[pl.ds(0, CHA)], acc_v.at[1, h], gsem[1]).wait()
    for h in range(2):
        pltpu.async_copy(acc_v.at[1, h], store_dst(NSUP - 1, h), ssem[1])
    for h in range(2):
        pltpu.make_async_copy(
            acc_v.at[0, h], store_dst(NSUP - 2, h), ssem[0]).wait()
    for h in range(2):
        pltpu.make_async_copy(
            acc_v.at[1, h], store_dst(NSUP - 1, h), ssem[1]).wait()


@functools.partial(
    pl.kernel,
    out_type=jax.ShapeDtypeStruct((L, BQ), f32),
    mesh=_MESH,
    scratch_types=[
        pltpu.VMEM((CH,), i32),
        pltpu.VMEM((CH, DEG), i32),
        pltpu.VMEM((DEG, CH), i32),
        pltpu.VMEM((CH, L), f32),
        pltpu.VMEM((L, CH), f32),
        pltpu.SemaphoreType.DMA,
    ],
    compiler_params=pltpu.CompilerParams(use_tc_tiling_on_sc=False, needs_layout_passes=False),
)
def _out_kernel(nodes_hbm, neigh_hbm, z_hbm, out_hbm,
                qidx_v, nb_v, nbt_v, acc_v, outt_v, sem):
    wid = _wid()
    qstart = wid * QPW

    def chunk_body(c, _):
        qbase = pl.multiple_of(qstart + c * CH, CH)
        pltpu.sync_copy(nodes_hbm.at[pl.ds(qbase, CH)], qidx_v)
        pltpu.async_copy(neigh_hbm.at[qidx_v], nb_v, sem).wait()
        for j in range(DEG):
            col = jnp.full((L,), j, i32)
            for g in range(CH // L):
                rid = lax.iota(i32, L) + g * L
                nbt_v[j, pl.ds(g * L, L)] = plsc.load_gather(nb_v, [rid, col])
        _zero_rows(acc_v, CH)
        cps = [
            pltpu.async_copy(z_hbm.at[nbt_v.at[j]], acc_v, sem, add=True)
            for j in range(DEG)
        ]
        for cp in cps:
            cp.wait()
        for j in range(L):
            col = jnp.full((L,), j, i32)
            for g in range(CH // L):
                rid = lax.iota(i32, L) + g * L
                outt_v[j, pl.ds(g * L, L)] = jnp.maximum(
                    plsc.load_gather(acc_v, [rid, col]), 0.0)
        for j in range(L):
            pltpu.sync_copy(outt_v.at[j], out_hbm.at[j, pl.ds(qbase, CH)])
        return 0

    lax.fori_loop(0, C_CHUNKS, chunk_body, 0)


PK = 128 // L
NR8 = NP // PK
RB = 800
NB = NR8 // RB


def _tc_body(fp_ref, ag_ref, wa_ref, wb_ref, w2_ref, z_ref):
    h = jnp.maximum(
        jnp.dot(fp_ref[...], wa_ref[...], preferred_element_type=f32)
        + jnp.dot(ag_ref[...], wb_ref[...], preferred_element_type=f32),
        0.0)
    z_ref[...] = jnp.dot(h, w2_ref[...], preferred_element_type=f32)


def _encode(fpad, agg, wa_bd, wb_bd, w2_bd):
    z8 = pl.pallas_call(
        _tc_body,
        grid=(NB,),
        in_specs=[
            pl.BlockSpec((RB, 128), lambda i: (i, 0)),
            pl.BlockSpec((RB, 128), lambda i: (i, 0)),
            pl.BlockSpec((128, PK * D_H1), lambda i: (0, 0)),
            pl.BlockSpec((128, PK * D_H1), lambda i: (0, 0)),
            pl.BlockSpec((PK * D_H1, 128), lambda i: (0, 0)),
        ],
        out_specs=pl.BlockSpec((RB, 128), lambda i: (i, 0)),
        out_shape=jax.ShapeDtypeStruct((NR8, 128), f32),
    )(fpad.reshape(NR8, 128), agg.reshape(NR8, 128), wa_bd, wb_bd, w2_bd)
    return z8.reshape(NP, L)


def kernel(nodes, neigh, features, W1, W2):
    wa = jnp.zeros((L, D_H1), f32).at[:D_FEAT].set(W1[:, :D_FEAT].T)
    wb = jnp.zeros((L, D_H1), f32).at[:D_FEAT].set(W1[:, D_FEAT:].T / DEG)
    w2p = jnp.zeros((D_H1, L), f32).at[:, :N_OUT].set(W2.T / DEG)
    eye8 = jnp.eye(PK, dtype=f32)
    wa_bd = jnp.kron(eye8, wa)
    wb_bd = jnp.kron(eye8, wb)
    w2_bd = jnp.kron(eye8, w2p)
    fpad = jnp.pad(features.astype(f32),
                   ((0, NP - N_NODES), (0, L - D_FEAT)))
    agg = _agg_kernel(fpad, neigh.astype(i32))
    z = _encode(fpad, agg, wa_bd, wb_bd, w2_bd)
    outp = _out_kernel(nodes.astype(i32), neigh.astype(i32), z)
    return outp[:N_OUT]


# speedup vs baseline: 1.1907x; 1.1907x over previous
import functools

import jax
import jax.numpy as jnp
from jax import lax
from jax.experimental import pallas as pl
from jax.experimental.pallas import tpu as pltpu
from jax.experimental.pallas import tpu_sc as plsc

N_NODES = 100000
DEG = 16
BQ = 16384
D_FEAT = 9
D_H1 = 20
N_OUT = 15

NP = 102400
NC, NS = 2, 16
NW = NC * NS
L = 16

CH = 128
CHA = 200
A_CHUNKS = 16
NSUP = A_CHUNKS // 2
A_SPAN = CHA * A_CHUNKS
A_LAST_START = N_NODES - A_SPAN
A_STRIDE = 3128

QPW = BQ // NW
C_CHUNKS = QPW // CH

f32 = jnp.float32
i32 = jnp.int32

_MESH = plsc.VectorSubcoreMesh(
    core_axis_name="c", subcore_axis_name="s", num_cores=NC, num_subcores=NS)


def _wid():
    return lax.axis_index("s") * NC + lax.axis_index("c")


def _zero_rows(ref, n):
    def body(i, _):
        ref[i] = jnp.zeros((L,), f32)
        return 0
    lax.fori_loop(0, n, body, 0)


@functools.partial(
    pl.kernel,
    out_type=jax.ShapeDtypeStruct((NP, L), f32),
    mesh=_MESH,
    scratch_types=[
        pltpu.VMEM((2, 2, DEG, CHA), i32),
        pltpu.VMEM((2, 2, CHA, L), f32),
        pltpu.VMEM_SHARED((NP, L), f32),
        pltpu.SemaphoreType.DMA,
        pltpu.SemaphoreType.DMA,
        pltpu.SemaphoreType.DMA,
        pltpu.SemaphoreType.DMA,
        pltpu.SemaphoreType.DMA,
        pltpu.SemaphoreType.DMA,
    ],
    compiler_params=pltpu.CompilerParams(use_tc_tiling_on_sc=False, needs_layout_passes=False),
)
def _agg_kernel(fpad_hbm, neight_hbm, agg_hbm, idx_v, acc_v, ftab,
                lsem0, lsem1, gsem0, gsem1, ssem0, ssem1):
    lsem = (lsem0, lsem1)
    gsem = (gsem0, gsem1)
    ssem = (ssem0, ssem1)
    wid = _wid()
    start = pl.multiple_of(jnp.minimum(wid * A_STRIDE, A_LAST_START), 8)
    sid = lax.axis_index("s")
    trows = NP // NS
    tstart = pl.multiple_of(sid * trows, 8)
    pltpu.sync_copy(fpad_hbm.at[pl.ds(tstart, trows)],
                    ftab.at[pl.ds(tstart, trows)])
    plsc.subcore_barrier()

    def idx_src(g, h, j):
        base = pl.multiple_of(start + (2 * g + h) * CHA, 8)
        return neight_hbm.at[j, pl.ds(base, CHA)]

    def fire_idx(g, b):
        for h in range(2):
            for j in range(DEG):
                pltpu.async_copy(idx_src(g, h, j), idx_v.at[b, h, j], lsem[b])

    def drain_idx(g, b):
        for h in range(2):
            for j in range(DEG):
                pltpu.make_async_copy(
                    idx_src(g, h, j), idx_v.at[b, h, j], lsem[b]).wait()

    def store_dst(g, h):
        base = pl.multiple_of(start + (2 * g + h) * CHA, 8)
        return agg_hbm.at[pl.ds(base, CHA)]

    def super_body(g, b):
        b1 = 1 - b
        drain_idx(g, b)

        @pl.when(g >= 2)
        def _():
            for h in range(2):
                pltpu.make_async_copy(
                    acc_v.at[b, h], store_dst(g - 2, h), ssem[b]).wait()
        for h in range(2):
            def zb(i, _, h=h):
                acc_v[b, h, i] = jnp.zeros((L,), f32)
                return 0
            lax.fori_loop(0, CHA, zb, 0)
        for h in range(2):
            for j in range(DEG):
                pltpu.async_copy(
                    ftab.at[idx_v.at[b, h, j]], acc_v.at[b, h], gsem[b],
                    add=True)

        @pl.when(g >= 1)
        def _():
            for h in range(2):
                for j in range(DEG):
                    pltpu.make_async_copy(
                        agg_hbm.at[pl.ds(0, CHA)], acc_v.at[b1, h],
                        gsem[b1]).wait()
            for h in range(2):
                pltpu.async_copy(acc_v.at[b1, h], store_dst(g - 1, h),
                                 ssem[b1])

        @pl.when(g <= NSUP - 2)
        def _():
            fire_idx(g + 1, b1)

    fire_idx(0, 0)

    def pair_body(k, _):
        super_body(2 * k, 0)
        super_body(2 * k + 1, 1)
        return 0

    lax.fori_loop(0, NSUP // 2, pair_body, 0)
    for h in range(2):
        for j in range(DEG):
            pltpu.make_async_copy(
                agg_hbm.at[pl.ds(0, CHA)], acc_v.at[1, h], gsem[1]).wait()
    for h in range(2):
        pltpu.async_copy(acc_v.at[1, h], store_dst(NSUP - 1, h), ssem[1])
    for h in range(2):
        pltpu.make_async_copy(
            acc_v.at[0, h], store_dst(NSUP - 2, h), ssem[0]).wait()
    for h in range(2):
        pltpu.make_async_copy(
            acc_v.at[1, h], store_dst(NSUP - 1, h), ssem[1]).wait()


@functools.partial(
    pl.kernel,
    out_type=jax.ShapeDtypeStruct((L, BQ), f32),
    mesh=_MESH,
    scratch_types=[
        pltpu.VMEM((CH,), i32),
        pltpu.VMEM((CH, DEG), i32),
        pltpu.VMEM((DEG, CH), i32),
        pltpu.VMEM((CH, L), f32),
        pltpu.VMEM((L, CH), f32),
        pltpu.SemaphoreType.DMA,
    ],
    compiler_params=pltpu.CompilerParams(use_tc_tiling_on_sc=False, needs_layout_passes=False),
)
def _out_kernel(nodes_hbm, neigh_hbm, z_hbm, out_hbm,
                qidx_v, nb_v, nbt_v, acc_v, outt_v, sem):
    wid = _wid()
    qstart = wid * QPW

    def chunk_body(c, _):
        qbase = pl.multiple_of(qstart + c * CH, CH)
        pltpu.sync_copy(nodes_hbm.at[pl.ds(qbase, CH)], qidx_v)
        pltpu.async_copy(neigh_hbm.at[qidx_v], nb_v, sem).wait()
        for j in range(DEG):
            col = jnp.full((L,), j, i32)
            for g in range(CH // L):
                rid = lax.iota(i32, L) + g * L
                nbt_v[j, pl.ds(g * L, L)] = plsc.load_gather(nb_v, [rid, col])
        _zero_rows(acc_v, CH)
        cps = [
            pltpu.async_copy(z_hbm.at[nbt_v.at[j]], acc_v, sem, add=True)
            for j in range(DEG)
        ]
        for cp in cps:
            cp.wait()
        for j in range(L):
            col = jnp.full((L,), j, i32)
            for g in range(CH // L):
                rid = lax.iota(i32, L) + g * L
                outt_v[j, pl.ds(g * L, L)] = jnp.maximum(
                    plsc.load_gather(acc_v, [rid, col]), 0.0)
        for j in range(L):
            pltpu.sync_copy(outt_v.at[j], out_hbm.at[j, pl.ds(qbase, CH)])
        return 0

    lax.fori_loop(0, C_CHUNKS, chunk_body, 0)


PK = 128 // L
NR8 = NP // PK
RB = 800
NB = NR8 // RB


def _tc_body(fp_ref, ag_ref, wa_ref, wb_ref, w2_ref, z_ref):
    h = jnp.maximum(
        jnp.dot(fp_ref[...], wa_ref[...], preferred_element_type=f32)
        + jnp.dot(ag_ref[...], wb_ref[...], preferred_element_type=f32),
        0.0)
    z_ref[...] = jnp.dot(h, w2_ref[...], preferred_element_type=f32)


def _encode(fpad, agg, wa_bd, wb_bd, w2_bd):
    z8 = pl.pallas_call(
        _tc_body,
        grid=(NB,),
        in_specs=[
            pl.BlockSpec((RB, 128), lambda i: (i, 0)),
            pl.BlockSpec((RB, 128), lambda i: (i, 0)),
            pl.BlockSpec((128, PK * D_H1), lambda i: (0, 0)),
            pl.BlockSpec((128, PK * D_H1), lambda i: (0, 0)),
            pl.BlockSpec((PK * D_H1, 128), lambda i: (0, 0)),
        ],
        out_specs=pl.BlockSpec((RB, 128), lambda i: (i, 0)),
        out_shape=jax.ShapeDtypeStruct((NR8, 128), f32),
    )(fpad.reshape(NR8, 128), agg.reshape(NR8, 128), wa_bd, wb_bd, w2_bd)
    return z8.reshape(NP, L)


def kernel(nodes, neigh, features, W1, W2):
    wa = jnp.zeros((L, D_H1), f32).at[:D_FEAT].set(W1[:, :D_FEAT].T)
    wb = jnp.zeros((L, D_H1), f32).at[:D_FEAT].set(W1[:, D_FEAT:].T / DEG)
    w2p = jnp.zeros((D_H1, L), f32).at[:, :N_OUT].set(W2.T / DEG)
    eye8 = jnp.eye(PK, dtype=f32)
    wa_bd = jnp.kron(eye8, wa)
    wb_bd = jnp.kron(eye8, wb)
    w2_bd = jnp.kron(eye8, w2p)
    fpad = jnp.pad(features.astype(f32),
                   ((0, NP - N_NODES), (0, L - D_FEAT)))
    neight = neigh.astype(i32).T
    agg = _agg_kernel(fpad, neight)
    z = _encode(fpad, agg, wa_bd, wb_bd, w2_bd)
    outp = _out_kernel(nodes.astype(i32), neigh.astype(i32), z)
    return outp[:N_OUT]


# speedup vs baseline: 1.1926x; 1.0016x over previous
import functools

import jax
import jax.numpy as jnp
from jax import lax
from jax.experimental import pallas as pl
from jax.experimental.pallas import tpu as pltpu
from jax.experimental.pallas import tpu_sc as plsc

N_NODES = 100000
DEG = 16
BQ = 16384
D_FEAT = 9
D_H1 = 20
N_OUT = 15

NP = 102400
NC, NS = 2, 16
NW = NC * NS
L = 16

CH = 128
CHA = 200
A_CHUNKS = 16
NSUP = A_CHUNKS // 2
A_SPAN = CHA * A_CHUNKS
A_LAST_START = N_NODES - A_SPAN
A_STRIDE = 3128

QPW = BQ // NW
C_CHUNKS = QPW // CH

f32 = jnp.float32
i32 = jnp.int32

_MESH = plsc.VectorSubcoreMesh(
    core_axis_name="c", subcore_axis_name="s", num_cores=NC, num_subcores=NS)


def _wid():
    return lax.axis_index("s") * NC + lax.axis_index("c")


def _zero_rows(ref, n):
    def body(i, _):
        ref[i] = jnp.zeros((L,), f32)
        return 0
    lax.fori_loop(0, n, body, 0)


@functools.partial(
    pl.kernel,
    out_type=jax.ShapeDtypeStruct((NP, L), f32),
    mesh=_MESH,
    scratch_types=[
        pltpu.VMEM((2, 2, DEG, CHA), i32),
        pltpu.VMEM((2, 2, CHA, L), f32),
        pltpu.VMEM_SHARED((NP, L), f32),
        pltpu.SemaphoreType.DMA,
        pltpu.SemaphoreType.DMA,
        pltpu.SemaphoreType.DMA,
        pltpu.SemaphoreType.DMA,
        pltpu.SemaphoreType.DMA,
        pltpu.SemaphoreType.DMA,
    ],
    compiler_params=pltpu.CompilerParams(use_tc_tiling_on_sc=False, needs_layout_passes=False),
)
def _agg_kernel(fpad_hbm, neight_hbm, agg_hbm, idx_v, acc_v, ftab,
                lsem0, lsem1, gsem0, gsem1, ssem0, ssem1):
    lsem = (lsem0, lsem1)
    gsem = (gsem0, gsem1)
    ssem = (ssem0, ssem1)
    wid = _wid()
    start = pl.multiple_of(jnp.minimum(wid * A_STRIDE, A_LAST_START), 8)
    sid = lax.axis_index("s")
    trows = NP // NS
    tstart = pl.multiple_of(sid * trows, 8)
    pltpu.sync_copy(fpad_hbm.at[pl.ds(tstart, trows)],
                    ftab.at[pl.ds(tstart, trows)])
    plsc.subcore_barrier()

    def idx_src(g, h, j):
        base = pl.multiple_of(start + (2 * g + h) * CHA, 8)
        return neight_hbm.at[j, pl.ds(base, CHA)]

    def fire_idx(g, b):
        for h in range(2):
            for j in range(DEG):
                pltpu.async_copy(idx_src(g, h, j), idx_v.at[b, h, j], lsem[b])

    def drain_idx(g, b):
        for h in range(2):
            for j in range(DEG):
                pltpu.make_async_copy(
                    idx_src(g, h, j), idx_v.at[b, h, j], lsem[b]).wait()

    def store_dst(g, h):
        base = pl.multiple_of(start + (2 * g + h) * CHA, 8)
        return agg_hbm.at[pl.ds(base, CHA)]

    def super_body(g, b):
        b1 = 1 - b
        drain_idx(g, b)

        @pl.when(g >= 2)
        def _():
            for h in range(2):
                pltpu.make_async_copy(
                    acc_v.at[b, h], store_dst(g - 2, h), ssem[b]).wait()
        for h in range(2):
            def zb(i, _, h=h):
                acc_v[b, h, i] = jnp.zeros((L,), f32)
                return 0
            lax.fori_loop(0, CHA, zb, 0)
        for h in range(2):
            for j in range(DEG):
                pltpu.async_copy(
                    ftab.at[idx_v.at[b, h, j]], acc_v.at[b, h], gsem[b],
                    add=True)

        @pl.when(g >= 1)
        def _():
            for h in range(2):
                for j in range(DEG):
                    pltpu.make_async_copy(
                        agg_hbm.at[pl.ds(0, CHA)], acc_v.at[b1, h],
                        gsem[b1]).wait()
            for h in range(2):
                pltpu.async_copy(acc_v.at[b1, h], store_dst(g - 1, h),
                                 ssem[b1])

        @pl.when(g <= NSUP - 2)
        def _():
            fire_idx(g + 1, b1)

    fire_idx(0, 0)

    def pair_body(k, _):
        super_body(2 * k, 0)
        super_body(2 * k + 1, 1)
        return 0

    lax.fori_loop(0, NSUP // 2, pair_body, 0)
    for h in range(2):
        for j in range(DEG):
            pltpu.make_async_copy(
                agg_hbm.at[pl.ds(0, CHA)], acc_v.at[1, h], gsem[1]).wait()
    for h in range(2):
        pltpu.async_copy(acc_v.at[1, h], store_dst(NSUP - 1, h), ssem[1])
    for h in range(2):
        pltpu.make_async_copy(
            acc_v.at[0, h], store_dst(NSUP - 2, h), ssem[0]).wait()
    for h in range(2):
        pltpu.make_async_copy(
            acc_v.at[1, h], store_dst(NSUP - 1, h), ssem[1]).wait()


@functools.partial(
    pl.kernel,
    out_type=jax.ShapeDtypeStruct((L, BQ), f32),
    mesh=_MESH,
    scratch_types=[
        pltpu.VMEM((CH,), i32),
        pltpu.VMEM((CH, DEG), i32),
        pltpu.VMEM((DEG, CH), i32),
        pltpu.VMEM((CH, L), f32),
        pltpu.VMEM((L, CH), f32),
        pltpu.SemaphoreType.DMA,
    ],
    compiler_params=pltpu.CompilerParams(use_tc_tiling_on_sc=False, needs_layout_passes=False),
)
def _out_kernel(nodes_hbm, neigh_hbm, z_hbm, out_hbm,
                qidx_v, nb_v, nbt_v, acc_v, outt_v, sem):
    wid = _wid()
    qstart = wid * QPW

    def chunk_body(c, _):
        qbase = pl.multiple_of(qstart + c * CH, CH)
        pltpu.sync_copy(nodes_hbm.at[pl.ds(qbase, CH)], qidx_v)
        pltpu.async_copy(neigh_hbm.at[qidx_v], nb_v, sem).wait()
        for j in range(DEG):
            col = jnp.full((L,), j, i32)
            for g in range(CH // L):
                rid = lax.iota(i32, L) + g * L
                nbt_v[j, pl.ds(g * L, L)] = plsc.load_gather(nb_v, [rid, col])
        _zero_rows(acc_v, CH)
        cps = [
            pltpu.async_copy(z_hbm.at[nbt_v.at[j]], acc_v, sem, add=True)
            for j in range(DEG)
        ]
        for cp in cps:
            cp.wait()
        for j in range(L):
            col = jnp.full((L,), j, i32)
            for g in range(CH // L):
                rid = lax.iota(i32, L) + g * L
                outt_v[j, pl.ds(g * L, L)] = jnp.maximum(
                    plsc.load_gather(acc_v, [rid, col]), 0.0)
        for j in range(L):
            pltpu.sync_copy(outt_v.at[j], out_hbm.at[j, pl.ds(qbase, CH)])
        return 0

    lax.fori_loop(0, C_CHUNKS, chunk_body, 0)


PK = 128 // L
NR8 = NP // PK
RB = 800
NB = NR8 // RB


def _tc_body(fp_ref, ag_ref, wa_ref, wb_ref, w2_ref, z_ref):
    h = jnp.maximum(
        jnp.dot(fp_ref[...], wa_ref[...], preferred_element_type=f32)
        + jnp.dot(ag_ref[...], wb_ref[...], preferred_element_type=f32),
        0.0)
    z_ref[...] = jnp.dot(h, w2_ref[...], preferred_element_type=f32)


def _encode(fpad, agg, wa_bd, wb_bd, w2_bd):
    z8 = pl.pallas_call(
        _tc_body,
        grid=(NB,),
        in_specs=[
            pl.BlockSpec((RB, 128), lambda i: (i, 0)),
            pl.BlockSpec((RB, 128), lambda i: (i, 0)),
            pl.BlockSpec((128, PK * D_H1), lambda i: (0, 0)),
            pl.BlockSpec((128, PK * D_H1), lambda i: (0, 0)),
            pl.BlockSpec((PK * D_H1, 128), lambda i: (0, 0)),
        ],
        out_specs=pl.BlockSpec((RB, 128), lambda i: (i, 0)),
        out_shape=jax.ShapeDtypeStruct((NR8, 128), f32),
    )(fpad.reshape(-1).reshape(NR8, 128), agg.reshape(-1).reshape(NR8, 128),
      wa_bd, wb_bd, w2_bd)
    return z8.reshape(NP, L)


def kernel(nodes, neigh, features, W1, W2):
    wa = jnp.zeros((L, D_H1), f32).at[:D_FEAT].set(W1[:, :D_FEAT].T)
    wb = jnp.zeros((L, D_H1), f32).at[:D_FEAT].set(W1[:, D_FEAT:].T / DEG)
    w2p = jnp.zeros((D_H1, L), f32).at[:, :N_OUT].set(W2.T / DEG)
    eye8 = jnp.eye(PK, dtype=f32)
    wa_bd = jnp.kron(eye8, wa)
    wb_bd = jnp.kron(eye8, wb)
    w2_bd = jnp.kron(eye8, w2p)
    fpad = jnp.pad(features.astype(f32),
                   ((0, NP - N_NODES), (0, L - D_FEAT)))
    neight = neigh.astype(i32).T
    agg = _agg_kernel(fpad, neight)
    z = _encode(fpad, agg, wa_bd, wb_bd, w2_bd)
    outp = _out_kernel(nodes.astype(i32), neigh.astype(i32), z)
    return outp[:N_OUT]
